# baseline (speedup 1.0000x reference)
import functools

import jax
import jax.numpy as jnp
from jax import lax
from jax.experimental import pallas as pl
from jax.experimental.pallas import tpu as pltpu
from jax.experimental.pallas import tpu_sc as plsc

N = 10000
E = 320000
D_FEAT = 128
D_EDGE = 16
D_HID = 300
CPAD = 320
TPAD = 384
CB = 64
NPASS = CPAD // CB
NUM_GRAPHS = 64
EPS = 1e-7
BN_EPS = 1e-5

NC = 2
NS = 16
NW = NC * NS
CH = 40
EPW = E // NW
NCHUNK = EPW // CH
RPW = 632
ROWS = RPW * NS

NB = 1000
NGRID = N // NB
EB = 4000
EGRID = E // EB


def _node_proj_body(x_ref, wsp_ref, bsp_ref, wd_ref, bd_ref,
                    t0_ref, t1_ref, t2_ref, xd_ref):
    xs = jnp.dot(x_ref[...], wsp_ref[...],
                 preferred_element_type=jnp.float32) + bsp_ref[...]
    t0_ref[...] = xs[:, 0:128]
    t1_ref[...] = xs[:, 128:256]
    t2_ref[...] = xs[:, 256:384]
    xd_ref[...] = jnp.dot(x_ref[...], wd_ref[...],
                          preferred_element_type=jnp.float32) + bd_ref[...]


def _node_proj(x, wsp, bsp, wd, bd):
    return pl.pallas_call(
        _node_proj_body,
        grid=(NGRID,),
        in_specs=[
            pl.BlockSpec((NB, D_FEAT), lambda i: (i, 0)),
            pl.BlockSpec((D_FEAT, TPAD), lambda i: (0, 0)),
            pl.BlockSpec((1, TPAD), lambda i: (0, 0)),
            pl.BlockSpec((D_FEAT, D_HID), lambda i: (0, 0)),
            pl.BlockSpec((1, D_HID), lambda i: (0, 0)),
        ],
        out_specs=[
            pl.BlockSpec((NB, 128), lambda i: (i, 0)),
            pl.BlockSpec((NB, 128), lambda i: (i, 0)),
            pl.BlockSpec((NB, 128), lambda i: (i, 0)),
            pl.BlockSpec((NB, D_HID), lambda i: (i, 0)),
        ],
        out_shape=[jax.ShapeDtypeStruct((N, 128), jnp.float32)] * 3
        + [jax.ShapeDtypeStruct((N, D_HID), jnp.float32)],
    )(x, wsp, bsp, wd, bd)


def _edge_proj_body(ea_ref, we_ref, be_ref, *out_refs):
    ep = jnp.dot(ea_ref[...], we_ref[...],
                 preferred_element_type=jnp.float32) + be_ref[...]
    for p, ref in enumerate(out_refs):
        ref[...] = ep[:, p * CB:(p + 1) * CB]


def _edge_proj(ea, wep, bep):
    return pl.pallas_call(
        _edge_proj_body,
        grid=(EGRID,),
        in_specs=[
            pl.BlockSpec((EB, D_EDGE), lambda i: (i, 0)),
            pl.BlockSpec((D_EDGE, CPAD), lambda i: (0, 0)),
            pl.BlockSpec((1, CPAD), lambda i: (0, 0)),
        ],
        out_specs=[pl.BlockSpec((EB, CB), lambda i: (i, 0))] * NPASS,
        out_shape=[jax.ShapeDtypeStruct((E, CB), jnp.float32)] * NPASS,
    )(ea, wep, bep)


def _sc_body(off, xs_hbm, ep_hbm, src_hbm, dst_hbm, zeros_hbm, out_hbm,
             sia, xg, eg, pq, di, acc, gsem, esem, ssem, dsem):
    c = lax.axis_index("c")
    s = lax.axis_index("s")
    wid = c * NS + s
    ebase = wid * EPW

    pltpu.sync_copy(src_hbm.at[pl.ds(ebase, EPW)], sia)
    pltpu.sync_copy(zeros_hbm, acc.at[pl.ds(s * RPW, RPW)])
    plsc.subcore_barrier()

    def start_loads(i):
        b = lax.rem(i, 2)
        b3 = lax.rem(i, 3)
        pltpu.async_copy(xs_hbm.at[sia.at[pl.ds(i * CH, CH)]],
                         xg.at[b], gsem.at[b])
        pltpu.async_copy(ep_hbm.at[pl.ds(ebase + i * CH, CH)],
                         eg.at[b], esem.at[b])
        pltpu.async_copy(dst_hbm.at[pl.ds(ebase + i * CH, CH)],
                         di.at[b3], dsem.at[b3])

    start_loads(0)

    def chunk_body(i, _):
        b = lax.rem(i, 2)
        b3 = lax.rem(i, 3)

        @pl.when(i >= 2)
        def _():
            pltpu.make_async_copy(pq.at[b], acc.at[di.at[b3]],
                                  ssem.at[b]).wait()

        @pl.when(i + 1 < NCHUNK)
        def _():
            start_loads(i + 1)

        pltpu.make_async_copy(xs_hbm.at[sia.at[pl.ds(i * CH, CH)]],
                              xg.at[b], gsem.at[b]).wait()
        pltpu.make_async_copy(ep_hbm.at[pl.ds(ebase + i * CH, CH)],
                              eg.at[b], esem.at[b]).wait()
        pltpu.make_async_copy(dst_hbm.at[pl.ds(ebase + i * CH, CH)],
                              di.at[b3], dsem.at[b3]).wait()

        def row_body(j, _):
            for k in range(CB // 16):
                a = xg[b, j, pl.ds(off + k * 16, 16)]
                e = eg[b, j, pl.ds(k * 16, 16)]
                m = jnp.maximum(a + e, 0.0)
                p = jnp.exp(m)
                pq[b, j, pl.ds(k * 16, 16)] = p
                pq[b, j, pl.ds(CB + k * 16, 16)] = p * m
            return _

        lax.fori_loop(0, CH, row_body, None)
        pltpu.async_copy(pq.at[b], acc.at[di.at[b3]], ssem.at[b], add=True)
        return _

    lax.fori_loop(0, NCHUNK, chunk_body, None)
    for i in (NCHUNK - 2, NCHUNK - 1):
        pltpu.make_async_copy(pq.at[i % 2], acc.at[di.at[i % 3]],
                              ssem.at[i % 2]).wait()
    plsc.subcore_barrier()

    @pl.when(s < NS - 1)
    def _():
        pltpu.sync_copy(acc.at[pl.ds(s * RPW, RPW)],
                        out_hbm.at[c, pl.ds(s * RPW, RPW)])

    @pl.when(s == NS - 1)
    def _():
        last = (NS - 1) * RPW
        pltpu.sync_copy(acc.at[pl.ds(last, N - last)],
                        out_hbm.at[c, pl.ds(last, N - last)])


@functools.lru_cache(maxsize=None)
def _make_sc_pass(off):
    return pl.kernel(
        functools.partial(_sc_body, off),
        out_type=jax.ShapeDtypeStruct((NC, N, 2 * CB), jnp.float32),
        mesh=plsc.VectorSubcoreMesh(core_axis_name="c", subcore_axis_name="s",
                                    num_cores=NC, num_subcores=NS),
        scratch_types=[
            pltpu.VMEM((EPW,), jnp.int32),
            pltpu.VMEM((2, CH, 128), jnp.float32),
            pltpu.VMEM((2, CH, CB), jnp.float32),
            pltpu.VMEM((2, CH, 2 * CB), jnp.float32),
            pltpu.VMEM((3, CH), jnp.int32),
            pltpu.VMEM_SHARED((ROWS, 2 * CB), jnp.float32),
            pltpu.SemaphoreType.DMA((2,)),
            pltpu.SemaphoreType.DMA((2,)),
            pltpu.SemaphoreType.DMA((2,)),
            pltpu.SemaphoreType.DMA((3,)),
        ],
    )


def _head_body(o0_ref, o1_ref, o2_ref, o3_ref, o4_ref, xd_ref, b3_ref,
               w1_ref, b1_ref, g1_ref, be1_ref,
               w2_ref, b2_ref, g2_ref, be2_ref,
               w3_ref, bb3_ref, wa_ref, ba_ref, wb_ref, bb_ref,
               out_ref, sums_ref, cnt_ref):
    i = pl.program_id(0)

    @pl.when(i == 0)
    def _():
        sums_ref[...] = jnp.zeros_like(sums_ref)
        cnt_ref[...] = jnp.zeros_like(cnt_ref)

    parts = [o0_ref[...], o1_ref[...], o2_ref[...], o3_ref[...], o4_ref[...]]
    merged = [p[0] + p[1] for p in parts]
    den = jnp.concatenate([m[:, :CB] for m in merged], axis=1)[:, :D_HID]
    num = jnp.concatenate([m[:, CB:] for m in merged], axis=1)[:, :D_HID]
    agg = num / (den + 1e-16) + xd_ref[...]

    h = jnp.dot(agg, w1_ref[...], preferred_element_type=jnp.float32) + b1_ref[...]
    h = jax.nn.relu(h * g1_ref[...] + be1_ref[...])
    h = jnp.dot(h, w2_ref[...], preferred_element_type=jnp.float32) + b2_ref[...]
    h = jax.nn.relu(h * g2_ref[...] + be2_ref[...])
    h = jnp.dot(h, w3_ref[...], preferred_element_type=jnp.float32) + bb3_ref[...]
    h = jax.nn.relu(jnp.dot(h, wa_ref[...], preferred_element_type=jnp.float32)
                    + ba_ref[...])
    h = jnp.dot(h, wb_ref[...], preferred_element_type=jnp.float32) + bb_ref[...]

    bvec = b3_ref[0, 0, :]
    onehot = (bvec[None, :] ==
              lax.broadcasted_iota(jnp.int32, (NUM_GRAPHS, NB), 0)
              ).astype(jnp.float32)
    sums_ref[...] += jnp.dot(onehot, h, preferred_element_type=jnp.float32)
    cnt_ref[...] += jnp.broadcast_to(jnp.sum(onehot, axis=1, keepdims=True),
                                     cnt_ref.shape)

    @pl.when(i == NGRID - 1)
    def _():
        out_ref[...] = sums_ref[...] / jnp.maximum(cnt_ref[...], 1.0)


def _head(outs, xd, batch3, w1, b1, g1i, be1, w2, b2, g2i, be2,
          w3, b3v, wa, ba, wb, bb):
    full = lambda shape: pl.BlockSpec(shape, lambda i: tuple(0 for _ in shape))
    return pl.pallas_call(
        _head_body,
        grid=(NGRID,),
        in_specs=[pl.BlockSpec((NC, NB, 2 * CB), lambda i: (0, i, 0))] * NPASS
        + [
            pl.BlockSpec((NB, D_HID), lambda i: (i, 0)),
            pl.BlockSpec((1, 1, NB), lambda i: (i, 0, 0)),
            full((D_HID, 600)), full((1, 600)), full((1, 600)), full((1, 600)),
            full((600, 600)), full((1, 600)), full((1, 600)), full((1, 600)),
            full((600, D_HID)), full((1, D_HID)),
            full((D_HID, 256)), full((1, 256)),
            full((256, 128)), full((1, 128)),
        ],
        out_specs=pl.BlockSpec((NUM_GRAPHS, 128), lambda i: (0, 0)),
        out_shape=jax.ShapeDtypeStruct((NUM_GRAPHS, 128), jnp.float32),
        scratch_shapes=[
            pltpu.VMEM((NUM_GRAPHS, 128), jnp.float32),
            pltpu.VMEM((NUM_GRAPHS, 128), jnp.float32),
        ],
    )(*outs, xd, batch3, w1, b1, g1i, be1, w2, b2, g2i, be2,
      w3, b3v, wa, ba, wb, bb)


def kernel(x, edge_index, edge_attr, batch,
           W_src, b_src, W_dst, b_dst, W_edge, b_edge,
           W1, b1, g1, be1, W2, b2, g2, be2, W3, b3,
           Wa, ba, Wb, bb):
    f32 = jnp.float32
    wsp = jnp.pad(W_src, ((0, 0), (0, TPAD - D_HID)))
    bsp = jnp.pad(b_src, (0, TPAD - D_HID)).reshape(1, TPAD)
    wep = jnp.pad(W_edge, ((0, 0), (0, CPAD - D_HID)))
    bep = jnp.pad(b_edge, (0, CPAD - D_HID)).reshape(1, CPAD)

    t0, t1, t2, xd = _node_proj(x, wsp, bsp, W_dst, b_dst.reshape(1, D_HID))
    slabs = _edge_proj(edge_attr, wep, bep)
    tables = (t0, t0, t1, t1, t2)

    src = edge_index[0]
    dst = edge_index[1]
    zeros = jnp.zeros((RPW, 2 * CB), f32)

    outs = [_make_sc_pass(64 * (p % 2))(tables[p], slabs[p], src, dst, zeros)
            for p in range(NPASS)]

    inv = 1.0 / jnp.sqrt(jnp.float32(1.0 + BN_EPS))
    batch3 = batch.reshape(NGRID, 1, NB)
    return _head(outs, xd, batch3,
                 W1, b1.reshape(1, 600), (g1 * inv).reshape(1, 600),
                 be1.reshape(1, 600),
                 W2, b2.reshape(1, 600), (g2 * inv).reshape(1, 600),
                 be2.reshape(1, 600),
                 W3, b3.reshape(1, D_HID),
                 Wa, ba.reshape(1, 256), Wb, bb.reshape(1, 128))


# speedup vs baseline: 2.3550x; 2.3550x over previous
import functools

import jax
import jax.numpy as jnp
from jax import lax
from jax.experimental import pallas as pl
from jax.experimental.pallas import tpu as pltpu
from jax.experimental.pallas import tpu_sc as plsc

N = 10000
E = 320000
D_FEAT = 128
D_EDGE = 16
D_HID = 300
CPAD = 320
TPAD = 384
CB = 64
NPASS = CPAD // CB
NUM_GRAPHS = 64
EPS = 1e-7
BN_EPS = 1e-5

NC = 2
NS = 16
NW = NC * NS
CH = 40
EPW = E // NW
NCHUNK = EPW // CH
RPW = 632
ROWS = RPW * NS

NB = 1000
NGRID = N // NB
EB = 4000
EGRID = E // EB


def _node_proj_body(x_ref, wsp_ref, bsp_ref, wd_ref, bd_ref,
                    t0_ref, t1_ref, t2_ref, xd_ref):
    xs = jnp.dot(x_ref[...], wsp_ref[...],
                 preferred_element_type=jnp.float32) + bsp_ref[...]
    t0_ref[...] = xs[:, 0:128]
    t1_ref[...] = xs[:, 128:256]
    t2_ref[...] = xs[:, 256:384]
    xd_ref[...] = jnp.dot(x_ref[...], wd_ref[...],
                          preferred_element_type=jnp.float32) + bd_ref[...]


def _node_proj(x, wsp, bsp, wd, bd):
    return pl.pallas_call(
        _node_proj_body,
        grid=(NGRID,),
        in_specs=[
            pl.BlockSpec((NB, D_FEAT), lambda i: (i, 0)),
            pl.BlockSpec((D_FEAT, TPAD), lambda i: (0, 0)),
            pl.BlockSpec((1, TPAD), lambda i: (0, 0)),
            pl.BlockSpec((D_FEAT, D_HID), lambda i: (0, 0)),
            pl.BlockSpec((1, D_HID), lambda i: (0, 0)),
        ],
        out_specs=[
            pl.BlockSpec((NB, 128), lambda i: (i, 0)),
            pl.BlockSpec((NB, 128), lambda i: (i, 0)),
            pl.BlockSpec((NB, 128), lambda i: (i, 0)),
            pl.BlockSpec((NB, D_HID), lambda i: (i, 0)),
        ],
        out_shape=[jax.ShapeDtypeStruct((N, 128), jnp.float32)] * 3
        + [jax.ShapeDtypeStruct((N, D_HID), jnp.float32)],
    )(x, wsp, bsp, wd, bd)


def _edge_proj_body(ea_ref, we_ref, be_ref, *out_refs):
    ep = jnp.dot(ea_ref[...], we_ref[...],
                 preferred_element_type=jnp.float32) + be_ref[...]
    for p, ref in enumerate(out_refs):
        ref[...] = ep[:, p * CB:(p + 1) * CB]


def _edge_proj(ea, wep, bep):
    return pl.pallas_call(
        _edge_proj_body,
        grid=(EGRID,),
        in_specs=[
            pl.BlockSpec((EB, D_EDGE), lambda i: (i, 0)),
            pl.BlockSpec((D_EDGE, CPAD), lambda i: (0, 0)),
            pl.BlockSpec((1, CPAD), lambda i: (0, 0)),
        ],
        out_specs=[pl.BlockSpec((EB, CB), lambda i: (i, 0))] * NPASS,
        out_shape=[jax.ShapeDtypeStruct((E, CB), jnp.float32)] * NPASS,
    )(ea, wep, bep)


def _sc_body(off, xs_hbm, ep_hbm, src_hbm, dst_hbm, zeros_hbm, out_hbm,
             sia, xg, eg, pq, di, acc, gsem, esem, ssem, dsem):
    c = lax.axis_index("c")
    s = lax.axis_index("s")
    wid = c * NS + s
    ebase = wid * EPW

    pltpu.sync_copy(src_hbm.at[pl.ds(ebase, EPW)], sia)
    pltpu.sync_copy(zeros_hbm, acc.at[pl.ds(s * RPW, RPW)])
    plsc.subcore_barrier()

    def start_data_loads(i):
        b = lax.rem(i, 2)
        pltpu.async_copy(xs_hbm.at[sia.at[pl.ds(i * CH, CH)]],
                         xg.at[b], gsem.at[b])
        pltpu.async_copy(ep_hbm.at[pl.ds(ebase + i * CH, CH)],
                         eg.at[b], esem.at[b])

    def start_di_load(i):
        b3 = lax.rem(i, 3)
        pltpu.async_copy(dst_hbm.at[pl.ds(ebase + i * CH, CH)],
                         di.at[b3], dsem.at[b3])

    start_data_loads(0)
    start_di_load(0)

    def chunk_body(i, _):
        b = lax.rem(i, 2)
        b3 = lax.rem(i, 3)

        @pl.when(i + 1 < NCHUNK)
        def _():
            start_data_loads(i + 1)

        @pl.when(i >= 2)
        def _():
            pltpu.make_async_copy(pq.at[b], acc.at[di.at[b3]],
                                  ssem.at[b]).wait()

        @pl.when(i + 1 < NCHUNK)
        def _():
            start_di_load(i + 1)

        pltpu.make_async_copy(xs_hbm.at[sia.at[pl.ds(i * CH, CH)]],
                              xg.at[b], gsem.at[b]).wait()
        pltpu.make_async_copy(ep_hbm.at[pl.ds(ebase + i * CH, CH)],
                              eg.at[b], esem.at[b]).wait()
        pltpu.make_async_copy(dst_hbm.at[pl.ds(ebase + i * CH, CH)],
                              di.at[b3], dsem.at[b3]).wait()

        @plsc.parallel_loop(0, CH, unroll=4)
        def _(j):
            for k in range(CB // 16):
                a = xg[b, j, pl.ds(off + k * 16, 16)]
                e = eg[b, j, pl.ds(k * 16, 16)]
                m = jnp.maximum(a + e, 0.0)
                p = jnp.exp(m)
                pq[b, j, pl.ds(k * 16, 16)] = p
                pq[b, j, pl.ds(CB + k * 16, 16)] = p * m

        pltpu.async_copy(pq.at[b], acc.at[di.at[b3]], ssem.at[b], add=True)
        return _

    lax.fori_loop(0, NCHUNK, chunk_body, None)
    for i in (NCHUNK - 2, NCHUNK - 1):
        pltpu.make_async_copy(pq.at[i % 2], acc.at[di.at[i % 3]],
                              ssem.at[i % 2]).wait()
    plsc.subcore_barrier()

    @pl.when(s < NS - 1)
    def _():
        pltpu.sync_copy(acc.at[pl.ds(s * RPW, RPW)],
                        out_hbm.at[c, pl.ds(s * RPW, RPW)])

    @pl.when(s == NS - 1)
    def _():
        last = (NS - 1) * RPW
        pltpu.sync_copy(acc.at[pl.ds(last, N - last)],
                        out_hbm.at[c, pl.ds(last, N - last)])


@functools.lru_cache(maxsize=None)
def _make_sc_pass(off):
    return pl.kernel(
        functools.partial(_sc_body, off),
        out_type=jax.ShapeDtypeStruct((NC, N, 2 * CB), jnp.float32),
        mesh=plsc.VectorSubcoreMesh(core_axis_name="c", subcore_axis_name="s",
                                    num_cores=NC, num_subcores=NS),
        scratch_types=[
            pltpu.VMEM((EPW,), jnp.int32),
            pltpu.VMEM((2, CH, 128), jnp.float32),
            pltpu.VMEM((2, CH, CB), jnp.float32),
            pltpu.VMEM((2, CH, 2 * CB), jnp.float32),
            pltpu.VMEM((3, CH), jnp.int32),
            pltpu.VMEM_SHARED((ROWS, 2 * CB), jnp.float32),
            pltpu.SemaphoreType.DMA((2,)),
            pltpu.SemaphoreType.DMA((2,)),
            pltpu.SemaphoreType.DMA((2,)),
            pltpu.SemaphoreType.DMA((3,)),
        ],
    )


def _head_body(o0_ref, o1_ref, o2_ref, o3_ref, o4_ref, xd_ref, b3_ref,
               w1_ref, b1_ref, g1_ref, be1_ref,
               w2_ref, b2_ref, g2_ref, be2_ref,
               w3_ref, bb3_ref, wa_ref, ba_ref, wb_ref, bb_ref,
               out_ref, sums_ref, cnt_ref):
    i = pl.program_id(0)

    @pl.when(i == 0)
    def _():
        sums_ref[...] = jnp.zeros_like(sums_ref)
        cnt_ref[...] = jnp.zeros_like(cnt_ref)

    parts = [o0_ref[...], o1_ref[...], o2_ref[...], o3_ref[...], o4_ref[...]]
    merged = [p[0] + p[1] for p in parts]
    den = jnp.concatenate([m[:, :CB] for m in merged], axis=1)[:, :D_HID]
    num = jnp.concatenate([m[:, CB:] for m in merged], axis=1)[:, :D_HID]
    agg = num / (den + 1e-16) + xd_ref[...]

    h = jnp.dot(agg, w1_ref[...], preferred_element_type=jnp.float32) + b1_ref[...]
    h = jax.nn.relu(h * g1_ref[...] + be1_ref[...])
    h = jnp.dot(h, w2_ref[...], preferred_element_type=jnp.float32) + b2_ref[...]
    h = jax.nn.relu(h * g2_ref[...] + be2_ref[...])
    h = jnp.dot(h, w3_ref[...], preferred_element_type=jnp.float32) + bb3_ref[...]
    h = jax.nn.relu(jnp.dot(h, wa_ref[...], preferred_element_type=jnp.float32)
                    + ba_ref[...])
    h = jnp.dot(h, wb_ref[...], preferred_element_type=jnp.float32) + bb_ref[...]

    bvec = b3_ref[0, 0, :]
    onehot = (bvec[None, :] ==
              lax.broadcasted_iota(jnp.int32, (NUM_GRAPHS, NB), 0)
              ).astype(jnp.float32)
    sums_ref[...] += jnp.dot(onehot, h, preferred_element_type=jnp.float32)
    cnt_ref[...] += jnp.broadcast_to(jnp.sum(onehot, axis=1, keepdims=True),
                                     cnt_ref.shape)

    @pl.when(i == NGRID - 1)
    def _():
        out_ref[...] = sums_ref[...] / jnp.maximum(cnt_ref[...], 1.0)


def _head(outs, xd, batch3, w1, b1, g1i, be1, w2, b2, g2i, be2,
          w3, b3v, wa, ba, wb, bb):
    full = lambda shape: pl.BlockSpec(shape, lambda i: tuple(0 for _ in shape))
    return pl.pallas_call(
        _head_body,
        grid=(NGRID,),
        in_specs=[pl.BlockSpec((NC, NB, 2 * CB), lambda i: (0, i, 0))] * NPASS
        + [
            pl.BlockSpec((NB, D_HID), lambda i: (i, 0)),
            pl.BlockSpec((1, 1, NB), lambda i: (i, 0, 0)),
            full((D_HID, 600)), full((1, 600)), full((1, 600)), full((1, 600)),
            full((600, 600)), full((1, 600)), full((1, 600)), full((1, 600)),
            full((600, D_HID)), full((1, D_HID)),
            full((D_HID, 256)), full((1, 256)),
            full((256, 128)), full((1, 128)),
        ],
        out_specs=pl.BlockSpec((NUM_GRAPHS, 128), lambda i: (0, 0)),
        out_shape=jax.ShapeDtypeStruct((NUM_GRAPHS, 128), jnp.float32),
        scratch_shapes=[
            pltpu.VMEM((NUM_GRAPHS, 128), jnp.float32),
            pltpu.VMEM((NUM_GRAPHS, 128), jnp.float32),
        ],
    )(*outs, xd, batch3, w1, b1, g1i, be1, w2, b2, g2i, be2,
      w3, b3v, wa, ba, wb, bb)


def kernel(x, edge_index, edge_attr, batch,
           W_src, b_src, W_dst, b_dst, W_edge, b_edge,
           W1, b1, g1, be1, W2, b2, g2, be2, W3, b3,
           Wa, ba, Wb, bb):
    f32 = jnp.float32
    wsp = jnp.pad(W_src, ((0, 0), (0, TPAD - D_HID)))
    bsp = jnp.pad(b_src, (0, TPAD - D_HID)).reshape(1, TPAD)
    wep = jnp.pad(W_edge, ((0, 0), (0, CPAD - D_HID)))
    bep = jnp.pad(b_edge, (0, CPAD - D_HID)).reshape(1, CPAD)

    t0, t1, t2, xd = _node_proj(x, wsp, bsp, W_dst, b_dst.reshape(1, D_HID))
    slabs = _edge_proj(edge_attr, wep, bep)
    tables = (t0, t0, t1, t1, t2)

    src = edge_index[0]
    dst = edge_index[1]
    zeros = jnp.zeros((RPW, 2 * CB), f32)

    outs = [_make_sc_pass(64 * (p % 2))(tables[p], slabs[p], src, dst, zeros)
            for p in range(NPASS)]

    inv = 1.0 / jnp.sqrt(jnp.float32(1.0 + BN_EPS))
    batch3 = batch.reshape(NGRID, 1, NB)
    return _head(outs, xd, batch3,
                 W1, b1.reshape(1, 600), (g1 * inv).reshape(1, 600),
                 be1.reshape(1, 600),
                 W2, b2.reshape(1, 600), (g2 * inv).reshape(1, 600),
                 be2.reshape(1, 600),
                 W3, b3.reshape(1, D_HID),
                 Wa, ba.reshape(1, 256), Wb, bb.reshape(1, 128))
